# Initial kernel scaffold; baseline (speedup 1.0000x reference)
import jax
import jax.numpy as jnp
from jax.experimental import pallas as pl


def kernel(x, edge_index, W1, b1, W2, b2, threshold):
    raise NotImplementedError("write your pallas kernel here")


# speedup vs baseline: 10.3493x; 10.3493x over previous
import dataclasses
import functools

import jax
import jax.numpy as jnp
from jax import lax
from jax.experimental import pallas as pl
from jax.experimental.pallas import tpu as pltpu
from jax.experimental.pallas import tpu_sc as plsc

N = 10000
E = 320000
D = 128
NP = 10240
EPAD = 327680
NC = 2
NS = 16
NW = NC * NS
EPW = EPAD // NW
NCH = EPW // 128
RPT = NP // NS
RB = 1024
GRID = NP // RB

_mesh = plsc.VectorSubcoreMesh(core_axis_name="c", subcore_axis_name="s")

_sc_params = pltpu.CompilerParams()
if "needs_layout_passes" in pltpu.CompilerParams.__dataclass_fields__:
    _sc_params = dataclasses.replace(_sc_params, needs_layout_passes=False)


@functools.partial(
    pl.kernel,
    mesh=_mesh,
    out_type=jax.ShapeDtypeStruct((NW, NP), jnp.float32),
    compiler_params=_sc_params,
    scratch_types=[
        pltpu.VMEM((EPW,), jnp.int32),
        pltpu.VMEM((NP,), jnp.float32),
    ],
)
def _deg_kernel(dst_hbm, out_hbm, dst_v, hist_v):
    c = lax.axis_index("c")
    s = lax.axis_index("s")
    wid = s * NC + c
    pltpu.sync_copy(dst_hbm.at[pl.ds(wid * EPW, EPW)], dst_v)

    zeros16 = jnp.zeros((16,), jnp.float32)
    ones16 = jnp.ones((16,), jnp.float32)

    @pl.loop(0, NP // 16)
    def _(i):
        hist_v[pl.ds(i * 16, 16)] = zeros16

    @pl.loop(0, EPW // 16)
    def _(j):
        idx = dst_v[pl.ds(j * 16, 16)]
        plsc.addupdate_scatter(hist_v, [idx], ones16)

    pltpu.sync_copy(hist_v, out_hbm.at[wid])


@functools.partial(
    pl.kernel,
    mesh=_mesh,
    out_type=jax.ShapeDtypeStruct((NC, NP, D), jnp.float32),
    scratch_types=[
        pltpu.VMEM((NCH, 128), jnp.int32),
        pltpu.VMEM((NCH, 128), jnp.int32),
        pltpu.VMEM((128, D), jnp.float32),
        pltpu.VMEM_SHARED((NP, D), jnp.float32),
        pltpu.SemaphoreType.DMA,
    ],
)
def _edge_kernel(g_hbm, src_hbm, dst_hbm, out_hbm, src_v, dst_v, rows_v,
                 acc_s, sem):
    c = lax.axis_index("c")
    s = lax.axis_index("s")
    wid = s * NC + c
    pltpu.sync_copy(src_hbm.at[pl.ds(wid * NCH, NCH)], src_v)
    pltpu.sync_copy(dst_hbm.at[pl.ds(wid * NCH, NCH)], dst_v)

    zeros16 = jnp.zeros((16,), jnp.float32)

    @pl.loop(0, 128)
    def _(r):
        @pl.loop(0, D // 16)
        def _(k):
            rows_v[r, pl.ds(k * 16, 16)] = zeros16

    @pl.loop(0, RPT // 128)
    def _(t):
        pltpu.sync_copy(rows_v, acc_s.at[pl.ds(s * RPT + t * 128, 128)])

    plsc.subcore_barrier()

    @pl.loop(0, NCH)
    def _(j):
        pltpu.async_copy(g_hbm.at[src_v.at[j]], rows_v, sem).wait()
        pltpu.sync_copy(rows_v, acc_s.at[dst_v.at[j]], add=True)

    plsc.subcore_barrier()

    @pl.loop(0, RPT // 128)
    def _(t):
        pltpu.sync_copy(acc_s.at[pl.ds(s * RPT + t * 128, 128)], rows_v)
        pltpu.sync_copy(rows_v, out_hbm.at[c, pl.ds(s * RPT + t * 128, 128)])


def _pre_body(x_ref, w_ref, hist_ref, g_ref, dinv_ref):
    deg = jnp.sum(hist_ref[...], axis=0) + 2.0
    dinv = lax.rsqrt(deg)
    h = jnp.dot(x_ref[...], w_ref[...], preferred_element_type=jnp.float32)
    g_ref[...] = h * dinv[:, None]
    dinv_ref[...] = dinv


_pre_call = pl.pallas_call(
    _pre_body,
    grid=(GRID,),
    in_specs=[
        pl.BlockSpec((RB, D), lambda i: (i, 0)),
        pl.BlockSpec((D, D), lambda i: (0, 0)),
        pl.BlockSpec((NW, RB), lambda i: (0, i)),
    ],
    out_specs=[
        pl.BlockSpec((RB, D), lambda i: (i, 0)),
        pl.BlockSpec((RB,), lambda i: (i,)),
    ],
    out_shape=[
        jax.ShapeDtypeStruct((NP, D), jnp.float32),
        jax.ShapeDtypeStruct((NP,), jnp.float32),
    ],
)


def _mid_body(p_ref, g1_ref, dinv_ref, b1_ref, w2_ref, g2_ref):
    acc = p_ref[0] + p_ref[1]
    dinv = dinv_ref[...]
    m = (acc + 2.0 * g1_ref[...]) * dinv[:, None] + b1_ref[...]
    h = jnp.maximum(m, 0.0)
    g2_ref[...] = jnp.dot(h, w2_ref[...],
                          preferred_element_type=jnp.float32) * dinv[:, None]


_mid_call = pl.pallas_call(
    _mid_body,
    grid=(GRID,),
    in_specs=[
        pl.BlockSpec((NC, RB, D), lambda i: (0, i, 0)),
        pl.BlockSpec((RB, D), lambda i: (i, 0)),
        pl.BlockSpec((RB,), lambda i: (i,)),
        pl.BlockSpec((1, D), lambda i: (0, 0)),
        pl.BlockSpec((D, D), lambda i: (0, 0)),
    ],
    out_specs=pl.BlockSpec((RB, D), lambda i: (i, 0)),
    out_shape=jax.ShapeDtypeStruct((NP, D), jnp.float32),
)


def _fin_body(p_ref, g2_ref, dinv_ref, b2_ref, out_ref):
    acc = p_ref[0] + p_ref[1]
    dinv = dinv_ref[...]
    m = (acc + 2.0 * g2_ref[...]) * dinv[:, None] + b2_ref[...]
    out_ref[...] = jnp.maximum(m, 0.0) + jnp.log1p(jnp.exp(-jnp.abs(m)))


_fin_call = pl.pallas_call(
    _fin_body,
    grid=(GRID,),
    in_specs=[
        pl.BlockSpec((NC, RB, D), lambda i: (0, i, 0)),
        pl.BlockSpec((RB, D), lambda i: (i, 0)),
        pl.BlockSpec((RB,), lambda i: (i,)),
        pl.BlockSpec((1, D), lambda i: (0, 0)),
    ],
    out_specs=pl.BlockSpec((RB, D), lambda i: (i, 0)),
    out_shape=jax.ShapeDtypeStruct((NP, D), jnp.float32),
)


def kernel(x, edge_index, W1, b1, W2, b2, threshold):
    ei = edge_index.astype(jnp.int32)
    pad = jnp.full((EPAD - E,), N, jnp.int32)
    src = jnp.concatenate([ei[0], pad])
    dst = jnp.concatenate([ei[1], pad])
    src2d = src.reshape(NW * NCH, 128)
    dst2d = dst.reshape(NW * NCH, 128)

    xp = jnp.pad(x, ((0, NP - N), (0, 0)))
    b1r = b1.reshape(1, D)
    b2r = b2.reshape(1, D)

    hist = _deg_kernel(dst)
    g1, dinv = _pre_call(xp, W1, hist)
    p1 = _edge_kernel(g1, src2d, dst2d)
    g2 = _mid_call(p1, g1, dinv, b1r, W2)
    p2 = _edge_kernel(g2, src2d, dst2d)
    out = _fin_call(p2, g2, dinv, b2r)
    return out[:N]


# speedup vs baseline: 11.9631x; 1.1559x over previous
import dataclasses
import functools

import jax
import jax.numpy as jnp
from jax import lax
from jax.experimental import pallas as pl
from jax.experimental.pallas import tpu as pltpu
from jax.experimental.pallas import tpu_sc as plsc

N = 10000
E = 320000
D = 128
NP = 10240
EPAD = 327680
NC = 2
NS = 16
NW = NC * NS
EPW = EPAD // NW
NCH = EPW // 128
RPT = NP // NS
RB = 1024
GRID = NP // RB

_mesh = plsc.VectorSubcoreMesh(core_axis_name="c", subcore_axis_name="s")

_sc_params = pltpu.CompilerParams()
if "needs_layout_passes" in pltpu.CompilerParams.__dataclass_fields__:
    _sc_params = dataclasses.replace(_sc_params, needs_layout_passes=False)


@functools.partial(
    pl.kernel,
    mesh=_mesh,
    out_type=jax.ShapeDtypeStruct((NW, NP), jnp.float32),
    compiler_params=_sc_params,
    scratch_types=[
        pltpu.VMEM((EPW,), jnp.int32),
        pltpu.VMEM((NP,), jnp.float32),
    ],
)
def _deg_kernel(dst_hbm, out_hbm, dst_v, hist_v):
    c = lax.axis_index("c")
    s = lax.axis_index("s")
    wid = s * NC + c
    pltpu.sync_copy(dst_hbm.at[pl.ds(wid * EPW, EPW)], dst_v)

    zeros16 = jnp.zeros((16,), jnp.float32)
    ones16 = jnp.ones((16,), jnp.float32)

    @pl.loop(0, NP // 16)
    def _(i):
        hist_v[pl.ds(i * 16, 16)] = zeros16

    @pl.loop(0, EPW // 16)
    def _(j):
        idx = dst_v[pl.ds(j * 16, 16)]
        plsc.addupdate_scatter(hist_v, [idx], ones16)

    pltpu.sync_copy(hist_v, out_hbm.at[wid])


KBUF = 2
IH = NCH // 2


@functools.partial(
    pl.kernel,
    mesh=_mesh,
    out_type=jax.ShapeDtypeStruct((NC, NP, D), jnp.float32),
    scratch_types=[
        pltpu.VMEM((IH, 128), jnp.int32),
        pltpu.VMEM((IH, 128), jnp.int32),
        pltpu.VMEM((KBUF * 128, D), jnp.float32),
        pltpu.VMEM_SHARED((NP, D), jnp.float32),
    ] + [pltpu.SemaphoreType.DMA] * KBUF,
)
def _edge_kernel(g_hbm, src_hbm, dst_hbm, out_hbm, src_v, dst_v, rows_v,
                 acc_s, *gsems):
    c = lax.axis_index("c")
    s = lax.axis_index("s")
    wid = s * NC + c

    def buf(k):
        return rows_v.at[pl.ds(k * 128, 128)]

    zeros16 = jnp.zeros((16,), jnp.float32)

    @pl.loop(0, 128)
    def _(r):
        @pl.loop(0, D // 16)
        def _(k):
            rows_v[r, pl.ds(k * 16, 16)] = zeros16

    @pl.loop(0, RPT // 128)
    def _(t):
        pltpu.sync_copy(buf(0), acc_s.at[pl.ds(s * RPT + t * 128, 128)])

    plsc.subcore_barrier()

    def start_gather(j, k):
        pltpu.async_copy(g_hbm.at[src_v.at[j]], buf(k), gsems[k])

    def finish_chunk(j, k):
        pltpu.make_async_copy(g_hbm.at[src_v.at[j]], buf(k), gsems[k]).wait()
        pltpu.sync_copy(buf(k), acc_s.at[dst_v.at[j]], add=True)

    for half in range(2):
        base = wid * NCH + half * IH
        pltpu.sync_copy(src_hbm.at[pl.ds(base, IH)], src_v)
        pltpu.sync_copy(dst_hbm.at[pl.ds(base, IH)], dst_v)

        for k in range(KBUF):
            start_gather(k, k)

        @pl.loop(0, IH // KBUF - 1)
        def _(t):
            for k in range(KBUF):
                j = t * KBUF + k
                finish_chunk(j, k)
                start_gather(j + KBUF, k)

        for k in range(KBUF):
            finish_chunk(IH - KBUF + k, k)

    plsc.subcore_barrier()

    @pl.loop(0, RPT // 128)
    def _(t):
        pltpu.sync_copy(acc_s.at[pl.ds(s * RPT + t * 128, 128)], buf(0))
        pltpu.sync_copy(buf(0), out_hbm.at[c, pl.ds(s * RPT + t * 128, 128)])


def _pre_body(x_ref, w_ref, hist_ref, g_ref, dinv_ref):
    deg = jnp.sum(hist_ref[...], axis=0) + 2.0
    dinv = lax.rsqrt(deg)
    h = jnp.dot(x_ref[...], w_ref[...], preferred_element_type=jnp.float32)
    g_ref[...] = h * dinv[:, None]
    dinv_ref[...] = dinv


_pre_call = pl.pallas_call(
    _pre_body,
    grid=(GRID,),
    in_specs=[
        pl.BlockSpec((RB, D), lambda i: (i, 0)),
        pl.BlockSpec((D, D), lambda i: (0, 0)),
        pl.BlockSpec((NW, RB), lambda i: (0, i)),
    ],
    out_specs=[
        pl.BlockSpec((RB, D), lambda i: (i, 0)),
        pl.BlockSpec((RB,), lambda i: (i,)),
    ],
    out_shape=[
        jax.ShapeDtypeStruct((NP, D), jnp.float32),
        jax.ShapeDtypeStruct((NP,), jnp.float32),
    ],
)


def _mid_body(p_ref, g1_ref, dinv_ref, b1_ref, w2_ref, g2_ref):
    acc = p_ref[0] + p_ref[1]
    dinv = dinv_ref[...]
    m = (acc + 2.0 * g1_ref[...]) * dinv[:, None] + b1_ref[...]
    h = jnp.maximum(m, 0.0)
    g2_ref[...] = jnp.dot(h, w2_ref[...],
                          preferred_element_type=jnp.float32) * dinv[:, None]


_mid_call = pl.pallas_call(
    _mid_body,
    grid=(GRID,),
    in_specs=[
        pl.BlockSpec((NC, RB, D), lambda i: (0, i, 0)),
        pl.BlockSpec((RB, D), lambda i: (i, 0)),
        pl.BlockSpec((RB,), lambda i: (i,)),
        pl.BlockSpec((1, D), lambda i: (0, 0)),
        pl.BlockSpec((D, D), lambda i: (0, 0)),
    ],
    out_specs=pl.BlockSpec((RB, D), lambda i: (i, 0)),
    out_shape=jax.ShapeDtypeStruct((NP, D), jnp.float32),
)


def _fin_body(p_ref, g2_ref, dinv_ref, b2_ref, out_ref):
    acc = p_ref[0] + p_ref[1]
    dinv = dinv_ref[...]
    m = (acc + 2.0 * g2_ref[...]) * dinv[:, None] + b2_ref[...]
    out_ref[...] = jnp.maximum(m, 0.0) + jnp.log1p(jnp.exp(-jnp.abs(m)))


_fin_call = pl.pallas_call(
    _fin_body,
    grid=(GRID,),
    in_specs=[
        pl.BlockSpec((NC, RB, D), lambda i: (0, i, 0)),
        pl.BlockSpec((RB, D), lambda i: (i, 0)),
        pl.BlockSpec((RB,), lambda i: (i,)),
        pl.BlockSpec((1, D), lambda i: (0, 0)),
    ],
    out_specs=pl.BlockSpec((RB, D), lambda i: (i, 0)),
    out_shape=jax.ShapeDtypeStruct((NP, D), jnp.float32),
)


def kernel(x, edge_index, W1, b1, W2, b2, threshold):
    ei = edge_index.astype(jnp.int32)
    pad = jnp.full((EPAD - E,), N, jnp.int32)
    src = jnp.concatenate([ei[0], pad])
    dst = jnp.concatenate([ei[1], pad])
    src2d = src.reshape(NW * NCH, 128)
    dst2d = dst.reshape(NW * NCH, 128)

    xp = jnp.pad(x, ((0, NP - N), (0, 0)))
    b1r = b1.reshape(1, D)
    b2r = b2.reshape(1, D)

    hist = _deg_kernel(dst)
    g1, dinv = _pre_call(xp, W1, hist)
    p1 = _edge_kernel(g1, src2d, dst2d)
    g2 = _mid_call(p1, g1, dinv, b1r, W2)
    p2 = _edge_kernel(g2, src2d, dst2d)
    out = _fin_call(p2, g2, dinv, b2r)
    return out[:N]
